# Initial kernel scaffold; baseline (speedup 1.0000x reference)
import jax
import jax.numpy as jnp
from jax.experimental import pallas as pl


def kernel(input_ids, segment_ids, word_embeddings, segments_embedding):
    raise NotImplementedError("write your pallas kernel here")


# speedup vs baseline: 2.6168x; 2.6168x over previous
import functools

import jax
import jax.numpy as jnp
from jax import lax
from jax.experimental import pallas as pl
from jax.experimental.pallas import tpu as pltpu
from jax.experimental.pallas import tpu_sc as plsc

DIM = 64
LANES = 16
NUM_CORES = 2
NUM_SUBCORES = 16
NUM_WORKERS = NUM_CORES * NUM_SUBCORES
GATHER_ROWS = 128
K = 8
CHUNK = K * GATHER_ROWS


def _emb_fused(ids2, sids, word, seg, n_tokens):
  per_w = n_tokens // NUM_WORKERS
  n_chunks = per_w // CHUNK
  mesh = plsc.VectorSubcoreMesh(
      core_axis_name="c", subcore_axis_name="s",
      num_cores=NUM_CORES, num_subcores=NUM_SUBCORES)

  @functools.partial(
      pl.kernel,
      out_type=jax.ShapeDtypeStruct((n_tokens, DIM), jnp.float32),
      mesh=mesh,
      scratch_types=[
          pltpu.VMEM((K, GATHER_ROWS), jnp.int32),
          pltpu.VMEM((CHUNK,), jnp.int32),
          pltpu.VMEM((CHUNK, DIM), jnp.float32),
          pltpu.VMEM((2, DIM), jnp.float32),
          pltpu.SemaphoreType.DMA,
      ],
      compiler_params=pltpu.CompilerParams(use_tc_tiling_on_sc=False),
  )
  def body(ids_hbm, sids_hbm, word_hbm, seg_hbm, out_hbm,
           idx_v, sid_v, rows_v, seg_v, gsem):
    wid = lax.axis_index("s") * NUM_CORES + lax.axis_index("c")
    row_base = wid * per_w
    pltpu.sync_copy(seg_hbm, seg_v)
    s0 = [seg_v[0, pl.ds(LANES * j, LANES)] for j in range(DIM // LANES)]
    s1 = [seg_v[1, pl.ds(LANES * j, LANES)] for j in range(DIM // LANES)]
    sd = [a - b for a, b in zip(s1, s0)]

    def chunk_body(t, carry):
      base = row_base + t * CHUNK
      gbase = pl.multiple_of(base // GATHER_ROWS, 8)
      pltpu.sync_copy(ids_hbm.at[pl.ds(gbase, K)], idx_v)
      pltpu.sync_copy(sids_hbm.at[pl.ds(base, CHUNK)], sid_v)
      cps = [
          pltpu.async_copy(word_hbm.at[idx_v.at[j]],
                           rows_v.at[pl.ds(j * GATHER_ROWS, GATHER_ROWS)],
                           gsem)
          for j in range(K)
      ]
      for cp in cps:
        cp.wait()

      def group_body(g, c2):
        g16 = g * LANES
        sv = sid_v[pl.ds(g16, LANES)].astype(jnp.float32)
        for i in range(LANES):
          r = g16 + i
          fv = jnp.full((LANES,), sv[i], jnp.float32)
          for j in range(DIM // LANES):
            sl = pl.ds(LANES * j, LANES)
            rows_v[r, sl] = rows_v[r, sl] + (s0[j] + fv * sd[j])
        return c2

      lax.fori_loop(0, CHUNK // LANES, group_body, 0)
      pltpu.sync_copy(rows_v, out_hbm.at[pl.ds(base, CHUNK)])
      return carry

    lax.fori_loop(0, n_chunks, chunk_body, 0)

  return body(ids2, sids, word, seg)


def kernel(input_ids, segment_ids, word_embeddings, segments_embedding):
  b, s = input_ids.shape
  n = b * s
  ids2 = jnp.reshape(input_ids, (n // GATHER_ROWS, GATHER_ROWS))
  sids = jnp.reshape(segment_ids, (n,))
  out = _emb_fused(ids2, sids, word_embeddings, segments_embedding, n)
  return (jnp.reshape(out, (b, s, DIM)), None)
